# baseline (speedup 1.0000x reference)
import functools

import jax
import jax.numpy as jnp
from jax import lax
from jax.experimental import pallas as pl
from jax.experimental.pallas import tpu as pltpu
from jax.experimental.pallas import tpu_sc as plsc

NSUB = 16
NCORE = 2
G = 256
SUB = G // 128
DEPTH = 4
BLK = 2048
BPB = BLK // G


def _bcast16(v, i):
    return lax.gather(
        v,
        jnp.full((16, 1), i, jnp.int32),
        lax.GatherDimensionNumbers(
            offset_dims=(), collapsed_slice_dims=(0,), start_index_map=(0,)),
        (1,),
        mode=lax.GatherScatterMode.PROMISE_IN_BOUNDS)


def _make_prop(half_real, half_pad, ept, dh):
    npad = 2 * half_pad
    rows_pt = half_pad // NSUB
    nbat = ept // G
    nblk = ept // BLK
    mesh = plsc.VectorSubcoreMesh(core_axis_name="c", subcore_axis_name="s")

    @functools.partial(
        pl.kernel,
        mesh=mesh,
        out_type=(
            jax.ShapeDtypeStruct((npad, dh), jnp.float32),
            jax.ShapeDtypeStruct((npad, dh), jnp.float32),
        ),
        compiler_params=pltpu.CompilerParams(use_tc_tiling_on_sc=False),
        scratch_types=[
            pltpu.VMEM((2 * BLK,), jnp.int32),
            pltpu.VMEM((2 * BLK,), jnp.int32),
            pltpu.VMEM((2 * BLK,), jnp.float32),
            pltpu.VMEM((DEPTH * SUB, 128), jnp.int32),
            pltpu.VMEM((DEPTH * G, dh), jnp.float32),
            pltpu.VMEM_SHARED((half_pad, dh), jnp.float32),
            pltpu.SemaphoreType.DMA,
            pltpu.SemaphoreType.DMA,
            pltpu.SemaphoreType.DMA,
        ],
    )
    def prop(ta, tb, src_hbm, dst_hbm, val_hbm, z_hbm, outa, outb,
             src_v, dst_v, val_v, loc_v, rows_v, acc_sh, esem, gsem, ssem):
        c = lax.axis_index("c")
        s = lax.axis_index("s")
        base_node = c * half_real
        dummy_row = half_real + s
        t_base = s * ept

        for cp in range(2):
            t_hbm = ta if cp == 0 else tb
            out_hbm = outa if cp == 0 else outb

            pltpu.sync_copy(z_hbm, acc_sh.at[pl.ds(s * rows_pt, rows_pt)])
            plsc.subcore_barrier()

            def fire_block(k):
                off = t_base + k * BLK
                stg = (k % 2) * BLK
                pltpu.async_copy(src_hbm.at[pl.ds(off, BLK)],
                                 src_v.at[pl.ds(stg, BLK)], esem)
                pltpu.async_copy(dst_hbm.at[pl.ds(off, BLK)],
                                 dst_v.at[pl.ds(stg, BLK)], esem)
                pltpu.async_copy(val_hbm.at[pl.ds(off, BLK)],
                                 val_v.at[pl.ds(stg, BLK)], esem)

            def drain_block():
                for _ in range(3):
                    pltpu.make_async_copy(
                        src_hbm.at[pl.ds(0, BLK)],
                        src_v.at[pl.ds(0, BLK)], esem).wait()

            fire_block(0)

            def lbatch(bb, _):
                @pl.when((bb % BPB == 0) & (bb < nbat))
                def _():
                    drain_block()

                @pl.when((bb % BPB == 2) & (bb // BPB + 1 < nblk))
                def _():
                    fire_block(bb // BPB + 1)

                @pl.when(bb < nbat)
                def _():
                    @pl.when(bb >= DEPTH)
                    def _():
                        for _k in range(SUB):
                            pltpu.make_async_copy(
                                rows_v.at[pl.ds((bb % DEPTH) * G + _k * 128,
                                                128)],
                                acc_sh.at[pl.ds(0, 128)], ssem).wait()
                    stg = ((bb // BPB) % 2) * BLK + (bb % BPB) * G
                    for sb in range(SUB):
                        pltpu.async_copy(
                            t_hbm.at[src_v.at[pl.ds(stg + sb * 128, 128)]],
                            rows_v.at[pl.ds((bb % DEPTH) * G + sb * 128, 128)],
                            gsem)

                x = bb - 2

                @pl.when(bb >= 2)
                def _():
                    for _k in range(SUB):
                        pltpu.make_async_copy(
                            ta.at[pl.ds(0, 128)],
                            rows_v.at[pl.ds((x % DEPTH) * G + _k * 128, 128)],
                            gsem).wait()
                    xstg = ((x // BPB) % 2) * BLK + (x % BPB) * G
                    rbase = (x % DEPTH) * G

                    def group(g, _2):
                        e0 = xstg + g * 16
                        dst16 = dst_v[pl.ds(e0, 16)]
                        val16 = val_v[pl.ds(e0, 16)]
                        loc = dst16 - base_node
                        inb = (loc >= 0) & (loc < half_real)
                        loc_v[(x % DEPTH) * SUB + g // 8,
                              pl.ds((g % 8) * 16, 16)] = jnp.where(
                                  inb, loc, dummy_row)
                        for i in range(16):
                            bv = _bcast16(val16, i)
                            e = rbase + g * 16 + i
                            for jj in range(dh // 16):
                                rows_v[e, pl.ds(jj * 16, 16)] = (
                                    rows_v[e, pl.ds(jj * 16, 16)] * bv)
                        return 0

                    lax.fori_loop(0, G // 16, group, 0)
                    for sb in range(SUB):
                        pltpu.async_copy(
                            rows_v.at[pl.ds(rbase + sb * 128, 128)],
                            acc_sh.at[loc_v.at[(x % DEPTH) * SUB + sb]],
                            ssem, add=True)
                return 0

            lax.fori_loop(0, nbat + 2, lbatch, 0)

            for t in range(DEPTH):
                for _k in range(SUB):
                    pltpu.make_async_copy(
                        rows_v.at[pl.ds(t * G + _k * 128, 128)],
                        acc_sh.at[pl.ds(0, 128)], ssem).wait()

            plsc.subcore_barrier()
            pltpu.sync_copy(
                acc_sh.at[pl.ds(s * rows_pt, rows_pt)],
                out_hbm.at[pl.ds(c * half_pad + s * rows_pt, rows_pt)])
            plsc.subcore_barrier()

    return prop


def _make_gather_mean(npad, nidx, dh):
    per_tile = nidx // (NCORE * NSUB)
    idx_rows = per_tile // 128
    mesh = plsc.VectorSubcoreMesh(core_axis_name="c", subcore_axis_name="s")

    @functools.partial(
        pl.kernel,
        mesh=mesh,
        out_type=jax.ShapeDtypeStruct((nidx, 2 * dh), jnp.float32),
        compiler_params=pltpu.CompilerParams(use_tc_tiling_on_sc=False),
        scratch_types=[
            pltpu.VMEM((per_tile,), jnp.int32),
            pltpu.VMEM((8 * per_tile, dh), jnp.float32),
            pltpu.VMEM((per_tile, 2 * dh), jnp.float32),
            pltpu.SemaphoreType.DMA,
        ],
    )
    def gmean(t0a, t1a, t2a, t3a, t0b, t1b, t2b, t3b, gidx_hbm, out_hbm,
              idx_v, tbl_v, out_v, sem):
        c = lax.axis_index("c")
        s = lax.axis_index("s")
        wid = c * NSUB + s
        pltpu.sync_copy(gidx_hbm.at[pl.ds(wid * per_tile, per_tile)], idx_v)
        handles = []
        for k, t in enumerate((t0a, t1a, t2a, t3a, t0b, t1b, t2b, t3b)):
            for sb in range(idx_rows):
                handles.append(pltpu.async_copy(
                    t.at[idx_v.at[pl.ds(sb * 128, 128)]],
                    tbl_v.at[pl.ds(k * per_tile + sb * 128, 128)], sem))
        for h in handles:
            h.wait()

        def row(r, _):
            for half in range(2):
                o = 4 * half * per_tile
                for jj in range(dh // 16):
                    sl = pl.ds(jj * 16, 16)
                    acc = (tbl_v[o + r, sl]
                           + tbl_v[o + per_tile + r, sl]
                           + tbl_v[o + 2 * per_tile + r, sl]
                           + tbl_v[o + 3 * per_tile + r, sl])
                    out_v[r, pl.ds(half * dh + jj * 16, 16)] = acc * 0.25
            return 0

        lax.fori_loop(0, per_tile, row, 0)
        pltpu.sync_copy(out_v, out_hbm.at[pl.ds(wid * per_tile, per_tile)])

    return gmean


def _dense_body(ue_ref, ie_ref, wk_ref, bk_ref, ws_ref, out_ref):
    ue = ue_ref[...]
    ie = ie_ref[...]
    nf = wk_ref.shape[0]

    def factors(x):
        fs = []
        for k in range(nf):
            w = wk_ref[k]
            f = lax.dot_general(
                x, w, (((1,), (1,)), ((), ())),
                precision=lax.Precision.HIGHEST,
                preferred_element_type=jnp.float32)
            f = jnp.maximum(f + bk_ref[k][None, :], 0.0)
            n = jnp.sqrt(jnp.sum(f * f, axis=1, keepdims=True))
            fs.append(f / jnp.maximum(n, 1e-12))
        return fs

    uf = factors(ue)
    itf = factors(ie)
    acc = jnp.zeros((ue.shape[0],), jnp.float32)
    for i in range(nf):
        for j in range(nf):
            acc = acc + ws_ref[i, j] * jnp.sum(uf[i] * itf[j], axis=1)
    out_ref[...] = acc


def _dense_scores(ue, ie, wk, bk, ws):
    b, d = ue.shape
    bs = 512
    nf = wk.shape[0]
    return pl.pallas_call(
        _dense_body,
        grid=(b // bs,),
        in_specs=[
            pl.BlockSpec((bs, d), lambda i: (i, 0)),
            pl.BlockSpec((bs, d), lambda i: (i, 0)),
            pl.BlockSpec((nf, d, d), lambda i: (0, 0, 0)),
            pl.BlockSpec((nf, d), lambda i: (0, 0)),
            pl.BlockSpec((nf, nf), lambda i: (0, 0)),
        ],
        out_specs=pl.BlockSpec((bs,), lambda i: (i,)),
        out_shape=jax.ShapeDtypeStruct((b,), jnp.float32),
    )(ue, ie, wk, bk, ws)


def kernel(users, items, user_emb, item_emb, edge_index, edge_vals, Wk, bk, W_s):
    nu, d = user_emb.shape
    ni = item_emb.shape[0]
    e = edge_index.shape[1]
    bsz = users.shape[0]
    assert nu == ni and d % 32 == 0
    dh = d // 2
    half_real = nu
    half_pad = ((nu + NSUB + 127) // 128) * 128
    npad = 2 * half_pad
    gap = half_pad - half_real

    zpad = jnp.zeros((gap, d), jnp.float32)
    t0 = jnp.concatenate([user_emb, zpad, item_emb, zpad], axis=0)
    t0a = t0[:, :dh]
    t0b = t0[:, dh:]

    src = edge_index[0]
    dst = edge_index[1]
    src_p = src + jnp.where(src >= half_real, gap, 0).astype(jnp.int32)

    ept = ((e // NSUB + BLK - 1) // BLK) * BLK
    e_pad = ept * NSUB
    pad_n = e_pad - e
    src_p = jnp.pad(src_p, (0, pad_n))
    dst_p = jnp.pad(dst, (0, pad_n))
    val_p = jnp.pad(edge_vals, (0, pad_n))

    zrows = jnp.zeros((half_pad // NSUB, dh), jnp.float32)

    prop = _make_prop(half_real, half_pad, ept, dh)
    t1a, t1b = prop(t0a, t0b, src_p, dst_p, val_p, zrows)
    t2a, t2b = prop(t1a, t1b, src_p, dst_p, val_p, zrows)
    t3a, t3b = prop(t2a, t2b, src_p, dst_p, val_p, zrows)

    gidx = jnp.concatenate([users, items + half_pad])
    gmean = _make_gather_mean(npad, 2 * bsz, dh)
    ui = gmean(t0a, t1a, t2a, t3a, t0b, t1b, t2b, t3b, gidx)

    return _dense_scores(ui[:bsz], ui[bsz:], Wk, bk, W_s)


# speedup vs baseline: 1.0034x; 1.0034x over previous
import functools

import jax
import jax.numpy as jnp
from jax import lax
from jax.experimental import pallas as pl
from jax.experimental.pallas import tpu as pltpu
from jax.experimental.pallas import tpu_sc as plsc

NSUB = 16
NCORE = 2
G = 256
SUB = G // 128
DEPTH = 6
BLK = 2048
BPB = BLK // G


def _bcast16(v, i):
    return lax.gather(
        v,
        jnp.full((16, 1), i, jnp.int32),
        lax.GatherDimensionNumbers(
            offset_dims=(), collapsed_slice_dims=(0,), start_index_map=(0,)),
        (1,),
        mode=lax.GatherScatterMode.PROMISE_IN_BOUNDS)


def _make_prop(half_real, half_pad, ept, dh):
    npad = 2 * half_pad
    rows_pt = half_pad // NSUB
    nbat = ept // G
    nblk = ept // BLK
    mesh = plsc.VectorSubcoreMesh(core_axis_name="c", subcore_axis_name="s")

    @functools.partial(
        pl.kernel,
        mesh=mesh,
        out_type=(
            jax.ShapeDtypeStruct((npad, dh), jnp.float32),
            jax.ShapeDtypeStruct((npad, dh), jnp.float32),
        ),
        compiler_params=pltpu.CompilerParams(use_tc_tiling_on_sc=False),
        scratch_types=[
            pltpu.VMEM((2 * BLK,), jnp.int32),
            pltpu.VMEM((2 * BLK,), jnp.int32),
            pltpu.VMEM((2 * BLK,), jnp.float32),
            pltpu.VMEM((DEPTH * G,), jnp.int32),
            pltpu.VMEM((DEPTH * G, dh), jnp.float32),
            pltpu.VMEM_SHARED((half_pad, dh), jnp.float32),
            pltpu.SemaphoreType.DMA,
            pltpu.SemaphoreType.DMA,
            pltpu.SemaphoreType.DMA,
        ],
    )
    def prop(ta, tb, src_hbm, dst_hbm, val_hbm, z_hbm, outa, outb,
             src_v, dst_v, val_v, loc_v, rows_v, acc_sh, esem, gsem, ssem):
        c = lax.axis_index("c")
        s = lax.axis_index("s")
        base_node = c * half_real
        dummy_row = half_real + s
        t_base = s * ept

        for cp in range(2):
            t_hbm = ta if cp == 0 else tb
            out_hbm = outa if cp == 0 else outb

            pltpu.sync_copy(z_hbm, acc_sh.at[pl.ds(s * rows_pt, rows_pt)])
            plsc.subcore_barrier()

            def fire_block(k):
                off = t_base + k * BLK
                stg = (k % 2) * BLK
                pltpu.async_copy(src_hbm.at[pl.ds(off, BLK)],
                                 src_v.at[pl.ds(stg, BLK)], esem)
                pltpu.async_copy(dst_hbm.at[pl.ds(off, BLK)],
                                 dst_v.at[pl.ds(stg, BLK)], esem)
                pltpu.async_copy(val_hbm.at[pl.ds(off, BLK)],
                                 val_v.at[pl.ds(stg, BLK)], esem)

            def drain_block():
                for _ in range(3):
                    pltpu.make_async_copy(
                        src_hbm.at[pl.ds(0, BLK)],
                        src_v.at[pl.ds(0, BLK)], esem).wait()

            fire_block(0)

            def lbatch(bb, _):
                @pl.when((bb % BPB == 0) & (bb < nbat))
                def _():
                    drain_block()

                @pl.when((bb % BPB == 3) & (bb // BPB + 1 < nblk))
                def _():
                    fire_block(bb // BPB + 1)

                @pl.when(bb < nbat)
                def _():
                    @pl.when(bb >= DEPTH)
                    def _():
                        pltpu.make_async_copy(
                            rows_v.at[pl.ds((bb % DEPTH) * G, G)],
                            acc_sh.at[pl.ds(0, G)], ssem).wait()
                    stg = ((bb // BPB) % 2) * BLK + (bb % BPB) * G
                    pltpu.async_copy(
                        t_hbm.at[src_v.at[pl.ds(stg, G)]],
                        rows_v.at[pl.ds((bb % DEPTH) * G, G)], gsem)

                x = bb - 3

                @pl.when(bb >= 3)
                def _():
                    pltpu.make_async_copy(
                        ta.at[pl.ds(0, G)],
                        rows_v.at[pl.ds((x % DEPTH) * G, G)], gsem).wait()
                    xstg = ((x // BPB) % 2) * BLK + (x % BPB) * G
                    rbase = (x % DEPTH) * G

                    def group(g, _2):
                        e0 = xstg + g * 16
                        dst16 = dst_v[pl.ds(e0, 16)]
                        val16 = val_v[pl.ds(e0, 16)]
                        loc = dst16 - base_node
                        inb = (loc >= 0) & (loc < half_real)
                        loc_v[pl.ds((x % DEPTH) * G + g * 16, 16)] = (
                            jnp.where(inb, loc, dummy_row))
                        for i in range(16):
                            bv = _bcast16(val16, i)
                            e = rbase + g * 16 + i
                            for jj in range(dh // 16):
                                rows_v[e, pl.ds(jj * 16, 16)] = (
                                    rows_v[e, pl.ds(jj * 16, 16)] * bv)
                        return 0

                    lax.fori_loop(0, G // 16, group, 0)
                    pltpu.async_copy(
                        rows_v.at[pl.ds(rbase, G)],
                        acc_sh.at[loc_v.at[pl.ds((x % DEPTH) * G, G)]],
                        ssem, add=True)
                return 0

            lax.fori_loop(0, nbat + 3, lbatch, 0)

            for t in range(DEPTH):
                pltpu.make_async_copy(
                    rows_v.at[pl.ds(t * G, G)],
                    acc_sh.at[pl.ds(0, G)], ssem).wait()

            plsc.subcore_barrier()
            pltpu.sync_copy(
                acc_sh.at[pl.ds(s * rows_pt, rows_pt)],
                out_hbm.at[pl.ds(c * half_pad + s * rows_pt, rows_pt)])
            plsc.subcore_barrier()

    return prop


def _make_gather_mean(npad, nidx, dh):
    per_tile = nidx // (NCORE * NSUB)
    idx_rows = per_tile // 128
    mesh = plsc.VectorSubcoreMesh(core_axis_name="c", subcore_axis_name="s")

    @functools.partial(
        pl.kernel,
        mesh=mesh,
        out_type=jax.ShapeDtypeStruct((nidx, 2 * dh), jnp.float32),
        compiler_params=pltpu.CompilerParams(use_tc_tiling_on_sc=False),
        scratch_types=[
            pltpu.VMEM((per_tile,), jnp.int32),
            pltpu.VMEM((8 * per_tile, dh), jnp.float32),
            pltpu.VMEM((per_tile, 2 * dh), jnp.float32),
            pltpu.SemaphoreType.DMA,
        ],
    )
    def gmean(t0a, t1a, t2a, t3a, t0b, t1b, t2b, t3b, gidx_hbm, out_hbm,
              idx_v, tbl_v, out_v, sem):
        c = lax.axis_index("c")
        s = lax.axis_index("s")
        wid = c * NSUB + s
        pltpu.sync_copy(gidx_hbm.at[pl.ds(wid * per_tile, per_tile)], idx_v)
        handles = []
        for k, t in enumerate((t0a, t1a, t2a, t3a, t0b, t1b, t2b, t3b)):
            for sb in range(idx_rows):
                handles.append(pltpu.async_copy(
                    t.at[idx_v.at[pl.ds(sb * 128, 128)]],
                    tbl_v.at[pl.ds(k * per_tile + sb * 128, 128)], sem))
        for h in handles:
            h.wait()

        def row(r, _):
            for half in range(2):
                o = 4 * half * per_tile
                for jj in range(dh // 16):
                    sl = pl.ds(jj * 16, 16)
                    acc = (tbl_v[o + r, sl]
                           + tbl_v[o + per_tile + r, sl]
                           + tbl_v[o + 2 * per_tile + r, sl]
                           + tbl_v[o + 3 * per_tile + r, sl])
                    out_v[r, pl.ds(half * dh + jj * 16, 16)] = acc * 0.25
            return 0

        lax.fori_loop(0, per_tile, row, 0)
        pltpu.sync_copy(out_v, out_hbm.at[pl.ds(wid * per_tile, per_tile)])

    return gmean


def _dense_body(ue_ref, ie_ref, wk_ref, bk_ref, ws_ref, out_ref):
    ue = ue_ref[...]
    ie = ie_ref[...]
    nf = wk_ref.shape[0]

    def factors(x):
        fs = []
        for k in range(nf):
            w = wk_ref[k]
            f = lax.dot_general(
                x, w, (((1,), (1,)), ((), ())),
                precision=lax.Precision.HIGHEST,
                preferred_element_type=jnp.float32)
            f = jnp.maximum(f + bk_ref[k][None, :], 0.0)
            n = jnp.sqrt(jnp.sum(f * f, axis=1, keepdims=True))
            fs.append(f / jnp.maximum(n, 1e-12))
        return fs

    uf = factors(ue)
    itf = factors(ie)
    acc = jnp.zeros((ue.shape[0],), jnp.float32)
    for i in range(nf):
        for j in range(nf):
            acc = acc + ws_ref[i, j] * jnp.sum(uf[i] * itf[j], axis=1)
    out_ref[...] = acc


def _dense_scores(ue, ie, wk, bk, ws):
    b, d = ue.shape
    bs = 512
    nf = wk.shape[0]
    return pl.pallas_call(
        _dense_body,
        grid=(b // bs,),
        in_specs=[
            pl.BlockSpec((bs, d), lambda i: (i, 0)),
            pl.BlockSpec((bs, d), lambda i: (i, 0)),
            pl.BlockSpec((nf, d, d), lambda i: (0, 0, 0)),
            pl.BlockSpec((nf, d), lambda i: (0, 0)),
            pl.BlockSpec((nf, nf), lambda i: (0, 0)),
        ],
        out_specs=pl.BlockSpec((bs,), lambda i: (i,)),
        out_shape=jax.ShapeDtypeStruct((b,), jnp.float32),
    )(ue, ie, wk, bk, ws)


def kernel(users, items, user_emb, item_emb, edge_index, edge_vals, Wk, bk, W_s):
    nu, d = user_emb.shape
    ni = item_emb.shape[0]
    e = edge_index.shape[1]
    bsz = users.shape[0]
    assert nu == ni and d % 32 == 0
    dh = d // 2
    half_real = nu
    half_pad = ((nu + NSUB + 127) // 128) * 128
    npad = 2 * half_pad
    gap = half_pad - half_real

    zpad = jnp.zeros((gap, d), jnp.float32)
    t0 = jnp.concatenate([user_emb, zpad, item_emb, zpad], axis=0)
    t0a = t0[:, :dh]
    t0b = t0[:, dh:]

    src = edge_index[0]
    dst = edge_index[1]
    src_p = src + jnp.where(src >= half_real, gap, 0).astype(jnp.int32)

    ept = ((e // NSUB + BLK - 1) // BLK) * BLK
    e_pad = ept * NSUB
    pad_n = e_pad - e
    src_p = jnp.pad(src_p, (0, pad_n))
    dst_p = jnp.pad(dst, (0, pad_n))
    val_p = jnp.pad(edge_vals, (0, pad_n))

    zrows = jnp.zeros((half_pad // NSUB, dh), jnp.float32)

    prop = _make_prop(half_real, half_pad, ept, dh)
    t1a, t1b = prop(t0a, t0b, src_p, dst_p, val_p, zrows)
    t2a, t2b = prop(t1a, t1b, src_p, dst_p, val_p, zrows)
    t3a, t3b = prop(t2a, t2b, src_p, dst_p, val_p, zrows)

    gidx = jnp.concatenate([users, items + half_pad])
    gmean = _make_gather_mean(npad, 2 * bsz, dh)
    ui = gmean(t0a, t1a, t2a, t3a, t0b, t1b, t2b, t3b, gidx)

    return _dense_scores(ui[:bsz], ui[bsz:], Wk, bk, W_s)


# speedup vs baseline: 4.2722x; 4.2577x over previous
import functools

import jax
import jax.numpy as jnp
from jax import lax
from jax.experimental import pallas as pl
from jax.experimental.pallas import tpu as pltpu
from jax.experimental.pallas import tpu_sc as plsc

NSUB = 16
NCORE = 2
G = 128
DEPTH = 5
BLK = 1024
BPB = BLK // G


def _bcast16(v, i):
    return lax.gather(
        v,
        jnp.full((16, 1), i, jnp.int32),
        lax.GatherDimensionNumbers(
            offset_dims=(), collapsed_slice_dims=(0,), start_index_map=(0,)),
        (1,),
        mode=lax.GatherScatterMode.PROMISE_IN_BOUNDS)


def _make_prop(npad, ept, dh):
    rows_pt = npad // NSUB
    nbat = ept // G
    nblk = ept // BLK
    mesh = plsc.VectorSubcoreMesh(core_axis_name="c", subcore_axis_name="s")

    @functools.partial(
        pl.kernel,
        mesh=mesh,
        out_type=jax.ShapeDtypeStruct((2 * npad, dh), jnp.float32),
        compiler_params=pltpu.CompilerParams(use_tc_tiling_on_sc=False),
        scratch_types=[
            pltpu.VMEM((2 * BLK,), jnp.int32),
            pltpu.VMEM((2 * BLK,), jnp.int32),
            pltpu.VMEM((2 * BLK,), jnp.float32),
            pltpu.VMEM((DEPTH * G,), jnp.int32),
            pltpu.VMEM((DEPTH * G, dh), jnp.float32),
            pltpu.VMEM_SHARED((npad, dh), jnp.float32),
            pltpu.SemaphoreType.DMA,
            pltpu.SemaphoreType.DMA,
            pltpu.SemaphoreType.DMA,
        ],
    )
    def prop(t_hbm, src_hbm, dst_hbm, val_hbm, z_hbm, out_hbm,
             src_v, dst_v, val_v, loc_v, rows_v, acc_sh, esem, gsem, ssem):
        c = lax.axis_index("c")
        s = lax.axis_index("s")
        cbase = c * npad
        t_base = s * ept

        pltpu.sync_copy(z_hbm, acc_sh.at[pl.ds(s * rows_pt, rows_pt)])
        plsc.subcore_barrier()

        def fire_block(k):
            off = t_base + k * BLK
            stg = (k % 2) * BLK
            pltpu.async_copy(src_hbm.at[pl.ds(off, BLK)],
                             src_v.at[pl.ds(stg, BLK)], esem)
            pltpu.async_copy(dst_hbm.at[pl.ds(off, BLK)],
                             dst_v.at[pl.ds(stg, BLK)], esem)
            pltpu.async_copy(val_hbm.at[pl.ds(off, BLK)],
                             val_v.at[pl.ds(stg, BLK)], esem)

        def drain_block():
            for _ in range(3):
                pltpu.make_async_copy(
                    src_hbm.at[pl.ds(0, BLK)],
                    src_v.at[pl.ds(0, BLK)], esem).wait()

        fire_block(0)

        def lbatch(bb, _):
            @pl.when((bb % BPB == 0) & (bb < nbat))
            def _():
                drain_block()
                stg0 = ((bb // BPB) % 2) * BLK

                def addb(i, _2):
                    sl = pl.ds(stg0 + i * 16, 16)
                    src_v[sl] = src_v[sl] + cbase
                    return 0

                lax.fori_loop(0, BLK // 16, addb, 0)

            @pl.when((bb % BPB == 3) & (bb // BPB + 1 < nblk))
            def _():
                fire_block(bb // BPB + 1)

            @pl.when(bb < nbat)
            def _():
                @pl.when(bb >= DEPTH)
                def _():
                    pltpu.make_async_copy(
                        rows_v.at[pl.ds((bb % DEPTH) * G, G)],
                        acc_sh.at[pl.ds(0, G)], ssem).wait()
                stg = ((bb // BPB) % 2) * BLK + (bb % BPB) * G
                pltpu.async_copy(
                    t_hbm.at[src_v.at[pl.ds(stg, G)]],
                    rows_v.at[pl.ds((bb % DEPTH) * G, G)], gsem)

            x = bb - 3

            @pl.when(bb >= 3)
            def _():
                pltpu.make_async_copy(
                    t_hbm.at[pl.ds(0, G)],
                    rows_v.at[pl.ds((x % DEPTH) * G, G)], gsem).wait()
                xstg = ((x // BPB) % 2) * BLK + (x % BPB) * G
                rbase = (x % DEPTH) * G

                def group(g, _2):
                    e0 = xstg + g * 16
                    val16 = val_v[pl.ds(e0, 16)]
                    loc_v[pl.ds(rbase + g * 16, 16)] = dst_v[pl.ds(e0, 16)]
                    for i in range(16):
                        bv = _bcast16(val16, i)
                        e = rbase + g * 16 + i
                        for jj in range(dh // 16):
                            rows_v[e, pl.ds(jj * 16, 16)] = (
                                rows_v[e, pl.ds(jj * 16, 16)] * bv)
                    return 0

                lax.fori_loop(0, G // 16, group, 0)
                pltpu.async_copy(
                    rows_v.at[pl.ds(rbase, G)],
                    acc_sh.at[loc_v.at[pl.ds(rbase, G)]],
                    ssem, add=True)
            return 0

        lax.fori_loop(0, nbat + 3, lbatch, 0)

        for t in range(DEPTH):
            pltpu.make_async_copy(
                rows_v.at[pl.ds(t * G, G)],
                acc_sh.at[pl.ds(0, G)], ssem).wait()

        plsc.subcore_barrier()
        pltpu.sync_copy(
            acc_sh.at[pl.ds(s * rows_pt, rows_pt)],
            out_hbm.at[pl.ds(cbase + s * rows_pt, rows_pt)])
        plsc.subcore_barrier()

    return prop


def _make_gather_mean(npad, nidx, dh):
    per_tile = nidx // (NCORE * NSUB)
    idx_rows = per_tile // 128
    mesh = plsc.VectorSubcoreMesh(core_axis_name="c", subcore_axis_name="s")

    @functools.partial(
        pl.kernel,
        mesh=mesh,
        out_type=jax.ShapeDtypeStruct((nidx, 2 * dh), jnp.float32),
        compiler_params=pltpu.CompilerParams(use_tc_tiling_on_sc=False),
        scratch_types=[
            pltpu.VMEM((per_tile,), jnp.int32),
            pltpu.VMEM((per_tile,), jnp.int32),
            pltpu.VMEM((8 * per_tile, dh), jnp.float32),
            pltpu.VMEM((per_tile, 2 * dh), jnp.float32),
            pltpu.SemaphoreType.DMA,
        ],
    )
    def gmean(t0, t1, t2, t3, gidx_hbm, gidx2_hbm, out_hbm,
              idx_v, idx2_v, tbl_v, out_v, sem):
        c = lax.axis_index("c")
        s = lax.axis_index("s")
        wid = c * NSUB + s
        pltpu.sync_copy(gidx_hbm.at[pl.ds(wid * per_tile, per_tile)], idx_v)
        pltpu.sync_copy(gidx2_hbm.at[pl.ds(wid * per_tile, per_tile)], idx2_v)
        handles = []
        for k, t in enumerate((t0, t1, t2, t3)):
            for half, iv in enumerate((idx_v, idx2_v)):
                for sb in range(idx_rows):
                    handles.append(pltpu.async_copy(
                        t.at[iv.at[pl.ds(sb * 128, 128)]],
                        tbl_v.at[pl.ds((4 * half + k) * per_tile + sb * 128,
                                       128)], sem))
        for h in handles:
            h.wait()

        def row(r, _):
            for half in range(2):
                o = 4 * half * per_tile
                for jj in range(dh // 16):
                    sl = pl.ds(jj * 16, 16)
                    acc = (tbl_v[o + r, sl]
                           + tbl_v[o + per_tile + r, sl]
                           + tbl_v[o + 2 * per_tile + r, sl]
                           + tbl_v[o + 3 * per_tile + r, sl])
                    out_v[r, pl.ds(half * dh + jj * 16, 16)] = acc * 0.25
            return 0

        lax.fori_loop(0, per_tile, row, 0)
        pltpu.sync_copy(out_v, out_hbm.at[pl.ds(wid * per_tile, per_tile)])

    return gmean


def _dense_body(ue_ref, ie_ref, wk_ref, bk_ref, ws_ref, out_ref):
    ue = ue_ref[...]
    ie = ie_ref[...]
    nf = wk_ref.shape[0]

    def factors(x):
        fs = []
        for k in range(nf):
            w = wk_ref[k]
            f = lax.dot_general(
                x, w, (((1,), (1,)), ((), ())),
                precision=lax.Precision.HIGHEST,
                preferred_element_type=jnp.float32)
            f = jnp.maximum(f + bk_ref[k][None, :], 0.0)
            n = jnp.sqrt(jnp.sum(f * f, axis=1, keepdims=True))
            fs.append(f / jnp.maximum(n, 1e-12))
        return fs

    uf = factors(ue)
    itf = factors(ie)
    acc = jnp.zeros((ue.shape[0],), jnp.float32)
    for i in range(nf):
        for j in range(nf):
            acc = acc + ws_ref[i, j] * jnp.sum(uf[i] * itf[j], axis=1)
    out_ref[...] = acc


def _dense_scores(ue, ie, wk, bk, ws):
    b, d = ue.shape
    bs = 512
    nf = wk.shape[0]
    return pl.pallas_call(
        _dense_body,
        grid=(b // bs,),
        in_specs=[
            pl.BlockSpec((bs, d), lambda i: (i, 0)),
            pl.BlockSpec((bs, d), lambda i: (i, 0)),
            pl.BlockSpec((nf, d, d), lambda i: (0, 0, 0)),
            pl.BlockSpec((nf, d), lambda i: (0, 0)),
            pl.BlockSpec((nf, nf), lambda i: (0, 0)),
        ],
        out_specs=pl.BlockSpec((bs,), lambda i: (i,)),
        out_shape=jax.ShapeDtypeStruct((b,), jnp.float32),
    )(ue, ie, wk, bk, ws)


def kernel(users, items, user_emb, item_emb, edge_index, edge_vals, Wk, bk, W_s):
    nu, d = user_emb.shape
    ni = item_emb.shape[0]
    n = nu + ni
    e = edge_index.shape[1]
    bsz = users.shape[0]
    assert d % 32 == 0
    dh = d // 2
    npad = ((n + 127) // 128) * 128

    t0 = jnp.concatenate([user_emb, item_emb], axis=0)
    t0 = jnp.pad(t0, ((0, npad - n), (0, 0)))
    t0s = jnp.concatenate([t0[:, :dh], t0[:, dh:]], axis=0)

    src = edge_index[0]
    dst = edge_index[1]

    ept = ((e // NSUB + BLK - 1) // BLK) * BLK
    e_pad = ept * NSUB
    pad_n = e_pad - e
    src_p = jnp.pad(src, (0, pad_n))
    dst_p = jnp.pad(dst, (0, pad_n))
    val_p = jnp.pad(edge_vals, (0, pad_n))

    zrows = jnp.zeros((npad // NSUB, dh), jnp.float32)

    prop = _make_prop(npad, ept, dh)
    t1s = prop(t0s, src_p, dst_p, val_p, zrows)
    t2s = prop(t1s, src_p, dst_p, val_p, zrows)
    t3s = prop(t2s, src_p, dst_p, val_p, zrows)

    gidx = jnp.concatenate([users, items + nu])
    gidx2 = gidx + npad
    gmean = _make_gather_mean(npad, 2 * bsz, dh)
    ui = gmean(t0s, t1s, t2s, t3s, gidx, gidx2)

    return _dense_scores(ui[:bsz], ui[bsz:], Wk, bk, W_s)


# speedup vs baseline: 4.3370x; 1.0152x over previous
import functools

import jax
import jax.numpy as jnp
from jax import lax
from jax.experimental import pallas as pl
from jax.experimental.pallas import tpu as pltpu
from jax.experimental.pallas import tpu_sc as plsc

NSUB = 16
NCORE = 2
G = 128
DEPTH = 5
BLK = 1024
BPB = BLK // G


def _bcast16(v, i):
    return lax.gather(
        v,
        jnp.full((16, 1), i, jnp.int32),
        lax.GatherDimensionNumbers(
            offset_dims=(), collapsed_slice_dims=(0,), start_index_map=(0,)),
        (1,),
        mode=lax.GatherScatterMode.PROMISE_IN_BOUNDS)


def _make_prop(npad, ept, dh):
    rows_pt = npad // NSUB
    nbat = ept // G
    nblk = ept // BLK
    mesh = plsc.VectorSubcoreMesh(core_axis_name="c", subcore_axis_name="s")

    @functools.partial(
        pl.kernel,
        mesh=mesh,
        out_type=tuple(
            jax.ShapeDtypeStruct((2 * npad, dh), jnp.float32)
            for _ in range(3)),
        compiler_params=pltpu.CompilerParams(use_tc_tiling_on_sc=False),
        scratch_types=[
            pltpu.VMEM((2 * BLK,), jnp.int32),
            pltpu.VMEM((2 * BLK,), jnp.int32),
            pltpu.VMEM((2 * BLK,), jnp.float32),
            pltpu.VMEM((DEPTH * G,), jnp.int32),
            pltpu.VMEM((DEPTH * G, dh), jnp.float32),
            pltpu.VMEM_SHARED((npad, dh), jnp.float32),
            pltpu.SemaphoreType.DMA,
            pltpu.SemaphoreType.DMA,
            pltpu.SemaphoreType.DMA,
        ],
    )
    def prop(t0_hbm, src_hbm, dst_hbm, val_hbm, z_hbm, o1, o2, o3,
             src_v, dst_v, val_v, loc_v, rows_v, acc_sh, esem, gsem, ssem):
        c = lax.axis_index("c")
        s = lax.axis_index("s")
        cbase = c * npad
        t_base = s * ept

        for t_hbm, out_hbm in ((t0_hbm, o1), (o1, o2), (o2, o3)):
            pltpu.sync_copy(z_hbm, acc_sh.at[pl.ds(s * rows_pt, rows_pt)])
            plsc.subcore_barrier()

            def fire_block(k):
                off = t_base + k * BLK
                stg = (k % 2) * BLK
                pltpu.async_copy(src_hbm.at[pl.ds(off, BLK)],
                                 src_v.at[pl.ds(stg, BLK)], esem)
                pltpu.async_copy(dst_hbm.at[pl.ds(off, BLK)],
                                 dst_v.at[pl.ds(stg, BLK)], esem)
                pltpu.async_copy(val_hbm.at[pl.ds(off, BLK)],
                                 val_v.at[pl.ds(stg, BLK)], esem)

            def drain_block():
                for _ in range(3):
                    pltpu.make_async_copy(
                        src_hbm.at[pl.ds(0, BLK)],
                        src_v.at[pl.ds(0, BLK)], esem).wait()

            fire_block(0)

            def lbatch(bb, _, t_hbm=t_hbm):
                @pl.when((bb % BPB == 0) & (bb < nbat))
                def _():
                    drain_block()
                    stg0 = ((bb // BPB) % 2) * BLK

                    def addb(i, _2):
                        sl = pl.ds(stg0 + i * 16, 16)
                        src_v[sl] = src_v[sl] + cbase
                        return 0

                    lax.fori_loop(0, BLK // 16, addb, 0)

                @pl.when((bb % BPB == 3) & (bb // BPB + 1 < nblk))
                def _():
                    fire_block(bb // BPB + 1)

                @pl.when(bb < nbat)
                def _():
                    @pl.when(bb >= DEPTH)
                    def _():
                        pltpu.make_async_copy(
                            rows_v.at[pl.ds((bb % DEPTH) * G, G)],
                            acc_sh.at[pl.ds(0, G)], ssem).wait()
                    stg = ((bb // BPB) % 2) * BLK + (bb % BPB) * G
                    pltpu.async_copy(
                        t_hbm.at[src_v.at[pl.ds(stg, G)]],
                        rows_v.at[pl.ds((bb % DEPTH) * G, G)], gsem)

                x = bb - 3

                @pl.when(bb >= 3)
                def _():
                    pltpu.make_async_copy(
                        t_hbm.at[pl.ds(0, G)],
                        rows_v.at[pl.ds((x % DEPTH) * G, G)], gsem).wait()
                    xstg = ((x // BPB) % 2) * BLK + (x % BPB) * G
                    rbase = (x % DEPTH) * G

                    def group(g, _2):
                        e0 = xstg + g * 16
                        val16 = val_v[pl.ds(e0, 16)]
                        loc_v[pl.ds(rbase + g * 16, 16)] = (
                            dst_v[pl.ds(e0, 16)])
                        for i in range(16):
                            bv = _bcast16(val16, i)
                            e = rbase + g * 16 + i
                            for jj in range(dh // 16):
                                rows_v[e, pl.ds(jj * 16, 16)] = (
                                    rows_v[e, pl.ds(jj * 16, 16)] * bv)
                        return 0

                    lax.fori_loop(0, G // 16, group, 0)
                    pltpu.async_copy(
                        rows_v.at[pl.ds(rbase, G)],
                        acc_sh.at[loc_v.at[pl.ds(rbase, G)]],
                        ssem, add=True)
                return 0

            lax.fori_loop(0, nbat + 3, lbatch, 0)

            for t in range(DEPTH):
                pltpu.make_async_copy(
                    rows_v.at[pl.ds(t * G, G)],
                    acc_sh.at[pl.ds(0, G)], ssem).wait()

            plsc.subcore_barrier()
            pltpu.sync_copy(
                acc_sh.at[pl.ds(s * rows_pt, rows_pt)],
                out_hbm.at[pl.ds(cbase + s * rows_pt, rows_pt)])
            plsc.subcore_barrier()

    return prop


def _make_gather_mean(npad, nidx, dh):
    per_tile = nidx // (NCORE * NSUB)
    idx_rows = per_tile // 128
    mesh = plsc.VectorSubcoreMesh(core_axis_name="c", subcore_axis_name="s")

    @functools.partial(
        pl.kernel,
        mesh=mesh,
        out_type=jax.ShapeDtypeStruct((nidx, 2 * dh), jnp.float32),
        compiler_params=pltpu.CompilerParams(use_tc_tiling_on_sc=False),
        scratch_types=[
            pltpu.VMEM((per_tile,), jnp.int32),
            pltpu.VMEM((per_tile,), jnp.int32),
            pltpu.VMEM((8 * per_tile, dh), jnp.float32),
            pltpu.VMEM((per_tile, 2 * dh), jnp.float32),
            pltpu.SemaphoreType.DMA,
        ],
    )
    def gmean(t0, t1, t2, t3, gidx_hbm, gidx2_hbm, out_hbm,
              idx_v, idx2_v, tbl_v, out_v, sem):
        c = lax.axis_index("c")
        s = lax.axis_index("s")
        wid = c * NSUB + s
        pltpu.sync_copy(gidx_hbm.at[pl.ds(wid * per_tile, per_tile)], idx_v)
        pltpu.sync_copy(gidx2_hbm.at[pl.ds(wid * per_tile, per_tile)], idx2_v)
        handles = []
        for k, t in enumerate((t0, t1, t2, t3)):
            for half, iv in enumerate((idx_v, idx2_v)):
                for sb in range(idx_rows):
                    handles.append(pltpu.async_copy(
                        t.at[iv.at[pl.ds(sb * 128, 128)]],
                        tbl_v.at[pl.ds((4 * half + k) * per_tile + sb * 128,
                                       128)], sem))
        for h in handles:
            h.wait()

        def row(r, _):
            for half in range(2):
                o = 4 * half * per_tile
                for jj in range(dh // 16):
                    sl = pl.ds(jj * 16, 16)
                    acc = (tbl_v[o + r, sl]
                           + tbl_v[o + per_tile + r, sl]
                           + tbl_v[o + 2 * per_tile + r, sl]
                           + tbl_v[o + 3 * per_tile + r, sl])
                    out_v[r, pl.ds(half * dh + jj * 16, 16)] = acc * 0.25
            return 0

        lax.fori_loop(0, per_tile, row, 0)
        pltpu.sync_copy(out_v, out_hbm.at[pl.ds(wid * per_tile, per_tile)])

    return gmean


def _dense_body(ue_ref, ie_ref, wk_ref, bk_ref, ws_ref, out_ref):
    ue = ue_ref[...]
    ie = ie_ref[...]
    nf = wk_ref.shape[0]

    def factors(x):
        fs = []
        for k in range(nf):
            w = wk_ref[k]
            f = lax.dot_general(
                x, w, (((1,), (1,)), ((), ())),
                precision=lax.Precision.HIGHEST,
                preferred_element_type=jnp.float32)
            f = jnp.maximum(f + bk_ref[k][None, :], 0.0)
            n = jnp.sqrt(jnp.sum(f * f, axis=1, keepdims=True))
            fs.append(f / jnp.maximum(n, 1e-12))
        return fs

    uf = factors(ue)
    itf = factors(ie)
    acc = jnp.zeros((ue.shape[0],), jnp.float32)
    for i in range(nf):
        for j in range(nf):
            acc = acc + ws_ref[i, j] * jnp.sum(uf[i] * itf[j], axis=1)
    out_ref[...] = acc


def _dense_scores(ue, ie, wk, bk, ws):
    b, d = ue.shape
    bs = 512
    nf = wk.shape[0]
    return pl.pallas_call(
        _dense_body,
        grid=(b // bs,),
        in_specs=[
            pl.BlockSpec((bs, d), lambda i: (i, 0)),
            pl.BlockSpec((bs, d), lambda i: (i, 0)),
            pl.BlockSpec((nf, d, d), lambda i: (0, 0, 0)),
            pl.BlockSpec((nf, d), lambda i: (0, 0)),
            pl.BlockSpec((nf, nf), lambda i: (0, 0)),
        ],
        out_specs=pl.BlockSpec((bs,), lambda i: (i,)),
        out_shape=jax.ShapeDtypeStruct((b,), jnp.float32),
    )(ue, ie, wk, bk, ws)


def kernel(users, items, user_emb, item_emb, edge_index, edge_vals, Wk, bk, W_s):
    nu, d = user_emb.shape
    ni = item_emb.shape[0]
    n = nu + ni
    e = edge_index.shape[1]
    bsz = users.shape[0]
    assert d % 32 == 0
    dh = d // 2
    npad = ((n + 127) // 128) * 128

    t0 = jnp.concatenate([user_emb, item_emb], axis=0)
    t0 = jnp.pad(t0, ((0, npad - n), (0, 0)))
    t0s = jnp.concatenate([t0[:, :dh], t0[:, dh:]], axis=0)

    src = edge_index[0]
    dst = edge_index[1]

    ept = ((e // NSUB + BLK - 1) // BLK) * BLK
    e_pad = ept * NSUB
    pad_n = e_pad - e
    src_p = jnp.pad(src, (0, pad_n))
    dst_p = jnp.pad(dst, (0, pad_n))
    val_p = jnp.pad(edge_vals, (0, pad_n))

    zrows = jnp.zeros((npad // NSUB, dh), jnp.float32)

    prop = _make_prop(npad, ept, dh)
    t1s, t2s, t3s = prop(t0s, src_p, dst_p, val_p, zrows)

    gidx = jnp.concatenate([users, items + nu])
    gidx2 = gidx + npad
    gmean = _make_gather_mean(npad, 2 * bsz, dh)
    ui = gmean(t0s, t1s, t2s, t3s, gidx, gidx2)

    return _dense_scores(ui[:bsz], ui[bsz:], Wk, bk, W_s)
